# baseline (speedup 1.0000x reference)
import functools

import jax
import jax.numpy as jnp
from jax import lax
from jax.experimental import pallas as pl
from jax.experimental.pallas import tpu as pltpu
from jax.experimental.pallas import tpu_sc as plsc

BN, NN, MM = 8, 512, 64
S6, S8, A1, A2 = 1.0, 0.7875, 0.4289, 4.4407
K1, K3 = -16.0, -4.0
BOHR_INV = 1.0 / 0.52917721092
HALF_HARTREE = 13.605693122994
NREF = 25
ROWW = 128
NZ = 95
APT = 128
PPT = APT * MM
CHUNK = 128
NCHUNK = PPT // CHUNK


def _rsqrt(x):
    i = lax.bitcast_convert_type(x, jnp.int32)
    i = jnp.int32(0x5F3759DF) - lax.shift_right_arithmetic(i, 1)
    y = lax.bitcast_convert_type(i, jnp.float32)
    for _ in range(3):
        y = y * (1.5 - 0.5 * x * y * y)
    return y


_mesh = plsc.VectorSubcoreMesh(core_axis_name="c", subcore_axis_name="s")
_params = pltpu.CompilerParams(needs_layout_passes=False)


@functools.partial(
    pl.kernel,
    mesh=_mesh,
    out_type=jax.ShapeDtypeStruct((BN, NN), jnp.float32),
    compiler_params=_params,
    scratch_types=[
        pltpu.VMEM((NN,), jnp.float32),
        pltpu.VMEM((NN,), jnp.float32),
        pltpu.VMEM((NN,), jnp.float32),
        pltpu.VMEM((NN,), jnp.int32),
        pltpu.VMEM((NN,), jnp.float32),
        pltpu.VMEM((PPT,), jnp.int32),
        pltpu.VMEM((96,), jnp.float32),
        pltpu.VMEM((96,), jnp.float32),
        pltpu.VMEM((APT,), jnp.float32),
    ],
)
def _cn_kernel(cx, cy, cz, num, nbr, rcov96, cnmax96, cn_out,
               xv, yv, zv, numv, rcovn, nbrv, rtab, ctab, cnout):
    c = lax.axis_index("c")
    s = lax.axis_index("s")
    wid = c * 16 + s
    b = wid // 4
    i0 = (wid % 4) * APT

    pltpu.sync_copy(cx.at[b], xv)
    pltpu.sync_copy(cy.at[b], yv)
    pltpu.sync_copy(cz.at[b], zv)
    pltpu.sync_copy(num.at[b], numv)
    pltpu.sync_copy(nbr.at[b, pl.ds(i0 * MM, PPT)], nbrv)
    pltpu.sync_copy(rcov96, rtab)
    pltpu.sync_copy(cnmax96, ctab)

    iota = lax.iota(jnp.int32, 16)
    iota64 = iota * MM

    def fill_rcovn(t, carry):
        z16 = numv[pl.ds(t * 16, 16)]
        rcovn[pl.ds(t * 16, 16)] = plsc.load_gather(rtab, [z16])
        return carry

    lax.fori_loop(0, NN // 16, fill_rcovn, 0)

    def atom_body(a, carry):
        base = a * 16
        gbase = i0 + base
        ivec = gbase + iota
        xi = xv[pl.ds(gbase, 16)]
        yi = yv[pl.ds(gbase, 16)]
        zi = zv[pl.ds(gbase, 16)]
        rci = rcovn[pl.ds(gbase, 16)]

        def m_body(m, acc):
            idx = iota64 + (base * MM + m)
            j = plsc.load_gather(nbrv, [idx])
            xj = plsc.load_gather(xv, [j])
            yj = plsc.load_gather(yv, [j])
            zj = plsc.load_gather(zv, [j])
            rcj = plsc.load_gather(rcovn, [j])
            dx = xi - xj
            dy = yi - yj
            dz = zi - zj
            d2 = jnp.maximum(dx * dx + dy * dy + dz * dz, 1e-12)
            inv_d = _rsqrt(d2)
            t = K1 * ((rci + rcj) * inv_d * (1.0 / BOHR_INV) - 1.0)
            cnm = 1.0 / (1.0 + jnp.exp(t))
            return acc + jnp.where(j != ivec, cnm, 0.0)

        acc = lax.fori_loop(0, MM, m_body, jnp.zeros((16,), jnp.float32))
        z16 = numv[pl.ds(gbase, 16)]
        cm = plsc.load_gather(ctab, [z16])
        cnout[pl.ds(base, 16)] = jnp.minimum(acc, cm)
        return carry

    lax.fori_loop(0, APT // 16, atom_body, 0)
    pltpu.sync_copy(cnout, cn_out.at[b, pl.ds(i0, APT)])


@functools.partial(
    pl.kernel,
    mesh=_mesh,
    out_type=jax.ShapeDtypeStruct((BN, 64), jnp.float32),
    compiler_params=_params,
    scratch_types=[
        pltpu.VMEM((NN,), jnp.float32),
        pltpu.VMEM((NN,), jnp.float32),
        pltpu.VMEM((NN,), jnp.float32),
        pltpu.VMEM((NN,), jnp.int32),
        pltpu.VMEM((NN,), jnp.float32),
        pltpu.VMEM((NN,), jnp.float32),
        pltpu.VMEM((PPT,), jnp.int32),
        pltpu.VMEM((96,), jnp.float32),
        pltpu.VMEM((CHUNK,), jnp.int32),
        pltpu.VMEM((CHUNK,), jnp.int32),
        pltpu.VMEM((CHUNK, ROWW), jnp.float32),
        pltpu.VMEM((CHUNK, ROWW), jnp.float32),
        pltpu.VMEM((16,), jnp.float32),
        pltpu.SemaphoreType.DMA,
        pltpu.SemaphoreType.DMA,
    ],
)
def _energy_kernel(cx, cy, cz, num, nbr, cnarr, rr96, table, out,
                   xv, yv, zv, numv, cnv, rrv, nbrv, rtab, idxb0, idxb1,
                   rows0, rows1, accv, sem0, sem1):
    c = lax.axis_index("c")
    s = lax.axis_index("s")
    wid = c * 16 + s
    b = wid // 4
    i0 = (wid % 4) * APT

    pltpu.sync_copy(cx.at[b], xv)
    pltpu.sync_copy(cy.at[b], yv)
    pltpu.sync_copy(cz.at[b], zv)
    pltpu.sync_copy(num.at[b], numv)
    pltpu.sync_copy(cnarr.at[b], cnv)
    pltpu.sync_copy(nbr.at[b, pl.ds(i0 * MM, PPT)], nbrv)
    pltpu.sync_copy(rr96, rtab)

    iota = lax.iota(jnp.int32, 16)

    def fill_rrv(t, carry):
        z16 = numv[pl.ds(t * 16, 16)]
        rrv[pl.ds(t * 16, 16)] = plsc.load_gather(rtab, [z16])
        return carry

    lax.fori_loop(0, NN // 16, fill_rrv, 0)

    def fill_idx(ch, idxb):
        p0 = ch * CHUNK
        zivec = None
        for v in range(8):
            jv = nbrv[pl.ds(p0 + 16 * v, 16)]
            zj = plsc.load_gather(numv, [jv])
            if v % 4 == 0:
                ig = i0 + ch * 2 + v // 4
                zivec = plsc.load_gather(numv, [jnp.full((16,), ig, jnp.int32)])
            idxb[pl.ds(16 * v, 16)] = (zivec * NZ + zj) * 16 + iota

    def compute(ch, rows, acc):
        p0 = ch * CHUNK
        ig = None
        cni = xi = yi = zi = rri = None
        for v in range(8):
            jv = nbrv[pl.ds(p0 + 16 * v, 16)]
            if v % 4 == 0:
                ig = i0 + ch * 2 + v // 4
                igv = jnp.full((16,), ig, jnp.int32)
                cni = plsc.load_gather(cnv, [igv])
                xi = plsc.load_gather(xv, [igv])
                yi = plsc.load_gather(yv, [igv])
                zi = plsc.load_gather(zv, [igv])
                rri = plsc.load_gather(rrv, [igv])
            cnj = plsc.load_gather(cnv, [jv])
            xj = plsc.load_gather(xv, [jv])
            yj = plsc.load_gather(yv, [jv])
            zj = plsc.load_gather(zv, [jv])
            rrj = plsc.load_gather(rrv, [jv])
            rowid = (16 * v) + iota
            w = jnp.zeros((16,), jnp.float32)
            zacc = jnp.zeros((16,), jnp.float32)
            for e in range(NREF):
                c6 = plsc.load_gather(rows, [rowid, iota + e])
                ca = plsc.load_gather(rows, [rowid, iota + (e + 25)])
                cb = plsc.load_gather(rows, [rowid, iota + (e + 50)])
                da = cni - ca
                db = cnj - cb
                lv = jnp.exp(K3 * (da * da + db * db))
                w = w + lv
                zacc = zacc + c6 * lv
            mask = jnp.logical_and(jv != ig, w >= 1e-5)
            zacc = jnp.where(mask, zacc, 0.0)
            c6ij = zacc / jnp.maximum(w, 1e-5)
            dx = xi - xj
            dy = yi - yj
            dz = zi - zj
            d2 = jnp.maximum(dx * dx + dy * dy + dz * dz, 1e-12) * (BOHR_INV * BOHR_INV)
            d6 = d2 * d2 * d2
            d8 = d6 * d2
            rrij = 3.0 * rri * rrj
            r0 = A1 * (rrij * _rsqrt(rrij)) + A2
            r02 = r0 * r0
            r06 = r02 * r02 * r02
            r08 = r06 * r02
            acc = acc + c6ij * (S6 / (d6 + r06) + S8 * rrij / (d8 + r08))
        return acc

    NT = NCHUNK // 2
    fill_idx(0, idxb0)
    pltpu.async_copy(table.at[idxb0], rows0, sem0)

    def pair_body(t, acc):
        ch0 = 2 * t
        fill_idx(ch0 + 1, idxb1)
        pltpu.async_copy(table.at[idxb1], rows1, sem1)
        pltpu.make_async_copy(table.at[idxb0], rows0, sem0).wait()
        acc = compute(ch0, rows0, acc)

        @pl.when(t + 1 < NT)
        def _():
            fill_idx(ch0 + 2, idxb0)
            pltpu.async_copy(table.at[idxb0], rows0, sem0)

        pltpu.make_async_copy(table.at[idxb1], rows1, sem1).wait()
        return compute(ch0 + 1, rows1, acc)

    acc = lax.fori_loop(0, NT, pair_body, jnp.zeros((16,), jnp.float32))

    accv[...] = acc
    pltpu.sync_copy(accv, out.at[b, pl.ds((wid % 4) * 16, 16)])


def kernel(coord, c6ab, r4r2, rcov, cnmax, numbers, nbr_idx_lr):
    coord = coord.astype(jnp.float32)
    cx = coord[:, :, 0]
    cy = coord[:, :, 1]
    cz = coord[:, :, 2]
    num = numbers.astype(jnp.int32)
    nbr = nbr_idx_lr.astype(jnp.int32).reshape(BN, NN * MM)
    pad1 = jnp.zeros((1,), jnp.float32)
    rcov96 = jnp.concatenate([rcov.astype(jnp.float32), pad1])
    cnmax96 = jnp.concatenate([cnmax.astype(jnp.float32), pad1])
    rr96 = jnp.concatenate([r4r2.astype(jnp.float32), pad1])
    c6flat = c6ab.astype(jnp.float32).reshape(NZ * NZ, NREF, 3)
    vals = jnp.concatenate(
        [c6flat[:, :, 0], c6flat[:, :, 1], c6flat[:, :, 2]], axis=1)
    shift = jnp.arange(16, dtype=jnp.int32)[None, :, None]
    col = jnp.arange(ROWW, dtype=jnp.int32)[None, None, :]
    src = jnp.broadcast_to(jnp.clip(col - shift, 0, 3 * NREF - 1),
                           (NZ * NZ, 16, ROWW))
    vals_b = jnp.broadcast_to(vals[:, None, :], (NZ * NZ, 16, 3 * NREF))
    table = jnp.where(
        (col >= shift) & (col < shift + 3 * NREF),
        jnp.take_along_axis(vals_b, src, axis=2), 0.0)
    table = table.reshape(NZ * NZ * 16, ROWW)

    cn = _cn_kernel(cx, cy, cz, num, nbr, rcov96, cnmax96)
    partials = _energy_kernel(cx, cy, cz, num, nbr, cn, rr96, table)
    return jnp.sum(partials, axis=1) * (-HALF_HARTREE)


# speedup vs baseline: 1.5587x; 1.5587x over previous
import functools

import jax
import jax.numpy as jnp
from jax import lax
from jax.experimental import pallas as pl
from jax.experimental.pallas import tpu as pltpu
from jax.experimental.pallas import tpu_sc as plsc

BN, NN, MM = 8, 512, 64
S6, S8, A1, A2 = 1.0, 0.7875, 0.4289, 4.4407
K1, K3 = -16.0, -4.0
BOHR_INV = 1.0 / 0.52917721092
HALF_HARTREE = 13.605693122994
NREF = 25
ROWW = 128
NZ = 95
APT = 128
PPT = APT * MM
CHUNK = 128
NCHUNK = PPT // CHUNK


def _rsqrt(x):
    i = lax.bitcast_convert_type(x, jnp.int32)
    i = jnp.int32(0x5F3759DF) - lax.shift_right_arithmetic(i, 1)
    y = lax.bitcast_convert_type(i, jnp.float32)
    for _ in range(3):
        y = y * (1.5 - 0.5 * x * y * y)
    return y


_mesh = plsc.VectorSubcoreMesh(core_axis_name="c", subcore_axis_name="s")
_params = pltpu.CompilerParams(needs_layout_passes=False)


@functools.partial(
    pl.kernel,
    mesh=_mesh,
    out_type=jax.ShapeDtypeStruct((BN, NN), jnp.float32),
    compiler_params=_params,
    scratch_types=[
        pltpu.VMEM((NN,), jnp.float32),
        pltpu.VMEM((NN,), jnp.float32),
        pltpu.VMEM((NN,), jnp.float32),
        pltpu.VMEM((NN,), jnp.int32),
        pltpu.VMEM((NN,), jnp.float32),
        pltpu.VMEM((PPT,), jnp.int32),
        pltpu.VMEM((96,), jnp.float32),
        pltpu.VMEM((96,), jnp.float32),
        pltpu.VMEM((APT,), jnp.float32),
    ],
)
def _cn_kernel(cx, cy, cz, num, nbr, rcov96, cnmax96, cn_out,
               xv, yv, zv, numv, rcovn, nbrv, rtab, ctab, cnout):
    c = lax.axis_index("c")
    s = lax.axis_index("s")
    wid = c * 16 + s
    b = wid // 4
    i0 = (wid % 4) * APT

    pltpu.sync_copy(cx.at[b], xv)
    pltpu.sync_copy(cy.at[b], yv)
    pltpu.sync_copy(cz.at[b], zv)
    pltpu.sync_copy(num.at[b], numv)
    pltpu.sync_copy(nbr.at[b, pl.ds(i0 * MM, PPT)], nbrv)
    pltpu.sync_copy(rcov96, rtab)
    pltpu.sync_copy(cnmax96, ctab)

    iota = lax.iota(jnp.int32, 16)
    iota64 = iota * MM

    def fill_rcovn(t, carry):
        z16 = numv[pl.ds(t * 16, 16)]
        rcovn[pl.ds(t * 16, 16)] = plsc.load_gather(rtab, [z16])
        return carry

    lax.fori_loop(0, NN // 16, fill_rcovn, 0)

    def atom_body(a, carry):
        base = a * 16
        gbase = i0 + base
        ivec = gbase + iota
        xi = xv[pl.ds(gbase, 16)]
        yi = yv[pl.ds(gbase, 16)]
        zi = zv[pl.ds(gbase, 16)]
        rci = rcovn[pl.ds(gbase, 16)]

        def m_body(m, acc):
            idx = iota64 + (base * MM + m)
            j = plsc.load_gather(nbrv, [idx])
            xj = plsc.load_gather(xv, [j])
            yj = plsc.load_gather(yv, [j])
            zj = plsc.load_gather(zv, [j])
            rcj = plsc.load_gather(rcovn, [j])
            dx = xi - xj
            dy = yi - yj
            dz = zi - zj
            d2 = jnp.maximum(dx * dx + dy * dy + dz * dz, 1e-12)
            inv_d = _rsqrt(d2)
            t = K1 * ((rci + rcj) * inv_d * (1.0 / BOHR_INV) - 1.0)
            cnm = 1.0 / (1.0 + jnp.exp(t))
            return acc + jnp.where(j != ivec, cnm, 0.0)

        acc = lax.fori_loop(0, MM, m_body, jnp.zeros((16,), jnp.float32))
        z16 = numv[pl.ds(gbase, 16)]
        cm = plsc.load_gather(ctab, [z16])
        cnout[pl.ds(base, 16)] = jnp.minimum(acc, cm)
        return carry

    lax.fori_loop(0, APT // 16, atom_body, 0)
    pltpu.sync_copy(cnout, cn_out.at[b, pl.ds(i0, APT)])


@functools.partial(
    pl.kernel,
    mesh=_mesh,
    out_type=jax.ShapeDtypeStruct((BN, 64), jnp.float32),
    compiler_params=_params,
    scratch_types=[
        pltpu.VMEM((NN,), jnp.float32),
        pltpu.VMEM((NN,), jnp.float32),
        pltpu.VMEM((NN,), jnp.float32),
        pltpu.VMEM((NN,), jnp.int32),
        pltpu.VMEM((NN,), jnp.float32),
        pltpu.VMEM((NN,), jnp.float32),
        pltpu.VMEM((PPT,), jnp.int32),
        pltpu.VMEM((96,), jnp.float32),
        pltpu.VMEM((CHUNK,), jnp.int32),
        pltpu.VMEM((CHUNK,), jnp.int32),
        pltpu.VMEM((CHUNK, ROWW), jnp.float32),
        pltpu.VMEM((CHUNK, ROWW), jnp.float32),
        pltpu.VMEM((16,), jnp.float32),
        pltpu.SemaphoreType.DMA,
        pltpu.SemaphoreType.DMA,
    ],
)
def _energy_kernel(cx, cy, cz, num, nbr, cnarr, rr96, table, out,
                   xv, yv, zv, numv, cnv, rrv, nbrv, rtab, idxb0, idxb1,
                   rows0, rows1, accv, sem0, sem1):
    c = lax.axis_index("c")
    s = lax.axis_index("s")
    wid = c * 16 + s
    b = wid // 4
    i0 = (wid % 4) * APT

    pltpu.sync_copy(cx.at[b], xv)
    pltpu.sync_copy(cy.at[b], yv)
    pltpu.sync_copy(cz.at[b], zv)
    pltpu.sync_copy(num.at[b], numv)
    pltpu.sync_copy(cnarr.at[b], cnv)
    pltpu.sync_copy(nbr.at[b, pl.ds(i0 * MM, PPT)], nbrv)
    pltpu.sync_copy(rr96, rtab)

    iota = lax.iota(jnp.int32, 16)

    def fill_rrv(t, carry):
        z16 = numv[pl.ds(t * 16, 16)]
        rrv[pl.ds(t * 16, 16)] = plsc.load_gather(rtab, [z16])
        return carry

    lax.fori_loop(0, NN // 16, fill_rrv, 0)

    def fill_idx(ch, idxb):
        p0 = ch * CHUNK
        zivec = None
        for v in range(8):
            jv = nbrv[pl.ds(p0 + 16 * v, 16)]
            zj = plsc.load_gather(numv, [jv])
            if v % 4 == 0:
                ig = i0 + ch * 2 + v // 4
                zivec = plsc.load_gather(numv, [jnp.full((16,), ig, jnp.int32)])
            idxb[pl.ds(16 * v, 16)] = (zivec * NZ + zj) * 16 + iota

    def compute(ch, rows, acc):
        p0 = ch * CHUNK
        ig = None
        cni = xi = yi = zi = rri = None
        for v in range(8):
            jv = nbrv[pl.ds(p0 + 16 * v, 16)]
            if v % 4 == 0:
                ig = i0 + ch * 2 + v // 4
                igv = jnp.full((16,), ig, jnp.int32)
                cni = plsc.load_gather(cnv, [igv])
                xi = plsc.load_gather(xv, [igv])
                yi = plsc.load_gather(yv, [igv])
                zi = plsc.load_gather(zv, [igv])
                rri = plsc.load_gather(rrv, [igv])
            cnj = plsc.load_gather(cnv, [jv])
            xj = plsc.load_gather(xv, [jv])
            yj = plsc.load_gather(yv, [jv])
            zj = plsc.load_gather(zv, [jv])
            rrj = plsc.load_gather(rrv, [jv])
            rowid = (16 * v) + iota
            w = jnp.zeros((16,), jnp.float32)
            zacc = jnp.zeros((16,), jnp.float32)
            for e in range(NREF):
                c6 = plsc.load_gather(rows, [rowid, iota + e])
                ca = plsc.load_gather(rows, [rowid, iota + (e + 25)])
                cb = plsc.load_gather(rows, [rowid, iota + (e + 50)])
                da = cni - ca
                db = cnj - cb
                lv = jnp.exp(K3 * (da * da + db * db))
                w = w + lv
                zacc = zacc + c6 * lv
            mask = jnp.logical_and(jv != ig, w >= 1e-5)
            zacc = jnp.where(mask, zacc, 0.0)
            c6ij = zacc / jnp.maximum(w, 1e-5)
            dx = xi - xj
            dy = yi - yj
            dz = zi - zj
            d2 = jnp.maximum(dx * dx + dy * dy + dz * dz, 1e-12) * (BOHR_INV * BOHR_INV)
            d6 = d2 * d2 * d2
            d8 = d6 * d2
            rrij = 3.0 * rri * rrj
            r0 = A1 * (rrij * _rsqrt(rrij)) + A2
            r02 = r0 * r0
            r06 = r02 * r02 * r02
            r08 = r06 * r02
            acc = acc + c6ij * (S6 / (d6 + r06) + S8 * rrij / (d8 + r08))
        return acc

    NT = NCHUNK // 2
    fill_idx(0, idxb0)
    pltpu.async_copy(table.at[idxb0], rows0, sem0)

    def pair_body(t, acc):
        ch0 = 2 * t
        fill_idx(ch0 + 1, idxb1)
        pltpu.async_copy(table.at[idxb1], rows1, sem1)
        pltpu.make_async_copy(table.at[idxb0], rows0, sem0).wait()
        acc = compute(ch0, rows0, acc)

        @pl.when(t + 1 < NT)
        def _():
            fill_idx(ch0 + 2, idxb0)
            pltpu.async_copy(table.at[idxb0], rows0, sem0)

        pltpu.make_async_copy(table.at[idxb1], rows1, sem1).wait()
        return compute(ch0 + 1, rows1, acc)

    acc = lax.fori_loop(0, NT, pair_body, jnp.zeros((16,), jnp.float32))

    accv[...] = acc
    pltpu.sync_copy(accv, out.at[b, pl.ds((wid % 4) * 16, 16)])


def kernel(coord, c6ab, r4r2, rcov, cnmax, numbers, nbr_idx_lr):
    coord = coord.astype(jnp.float32)
    cx = coord[:, :, 0]
    cy = coord[:, :, 1]
    cz = coord[:, :, 2]
    num = numbers.astype(jnp.int32)
    nbr = nbr_idx_lr.astype(jnp.int32).reshape(BN, NN * MM)
    pad1 = jnp.zeros((1,), jnp.float32)
    rcov96 = jnp.concatenate([rcov.astype(jnp.float32), pad1])
    cnmax96 = jnp.concatenate([cnmax.astype(jnp.float32), pad1])
    rr96 = jnp.concatenate([r4r2.astype(jnp.float32), pad1])
    c6flat = c6ab.astype(jnp.float32).reshape(NZ * NZ, NREF, 3)
    vals = jnp.concatenate(
        [c6flat[:, :, 0], c6flat[:, :, 1], c6flat[:, :, 2]], axis=1)
    padded = jnp.concatenate(
        [vals, jnp.zeros((NZ * NZ, ROWW - 3 * NREF), jnp.float32)], axis=1)
    table = jnp.stack([jnp.roll(padded, s, axis=1) for s in range(16)],
                      axis=1).reshape(NZ * NZ * 16, ROWW)

    cn = _cn_kernel(cx, cy, cz, num, nbr, rcov96, cnmax96)
    partials = _energy_kernel(cx, cy, cz, num, nbr, cn, rr96, table)
    return jnp.sum(partials, axis=1) * (-HALF_HARTREE)


# speedup vs baseline: 10.1816x; 6.5320x over previous
import functools

import jax
import jax.numpy as jnp
from jax import lax
from jax.experimental import pallas as pl
from jax.experimental.pallas import tpu as pltpu
from jax.experimental.pallas import tpu_sc as plsc

BN, NN, MM = 8, 512, 64
S6, S8, A1, A2 = 1.0, 0.7875, 0.4289, 4.4407
K1, K3 = -16.0, -4.0
BOHR_INV = 1.0 / 0.52917721092
HALF_HARTREE = 13.605693122994
NREF = 25
ROWW = 128
NZ = 95
APT = 128
PPT = APT * MM
CHUNK = 128
NCHUNK = PPT // CHUNK


def _rsqrt(x):
    i = lax.bitcast_convert_type(x, jnp.int32)
    i = jnp.int32(0x5F3759DF) - lax.shift_right_arithmetic(i, 1)
    y = lax.bitcast_convert_type(i, jnp.float32)
    for _ in range(3):
        y = y * (1.5 - 0.5 * x * y * y)
    return y


_mesh = plsc.VectorSubcoreMesh(core_axis_name="c", subcore_axis_name="s")
_params = pltpu.CompilerParams(needs_layout_passes=False)


@functools.partial(
    pl.kernel,
    mesh=_mesh,
    out_type=jax.ShapeDtypeStruct((BN, NN), jnp.float32),
    compiler_params=_params,
    scratch_types=[
        pltpu.VMEM((NN,), jnp.float32),
        pltpu.VMEM((NN,), jnp.float32),
        pltpu.VMEM((NN,), jnp.float32),
        pltpu.VMEM((NN,), jnp.int32),
        pltpu.VMEM((NN,), jnp.float32),
        pltpu.VMEM((PPT,), jnp.int32),
        pltpu.VMEM((96,), jnp.float32),
        pltpu.VMEM((96,), jnp.float32),
        pltpu.VMEM((APT,), jnp.float32),
    ],
)
def _cn_kernel(cx, cy, cz, num, nbr, rcov96, cnmax96, cn_out,
               xv, yv, zv, numv, rcovn, nbrv, rtab, ctab, cnout):
    c = lax.axis_index("c")
    s = lax.axis_index("s")
    wid = c * 16 + s
    b = wid // 4
    i0 = (wid % 4) * APT

    pltpu.sync_copy(cx.at[b], xv)
    pltpu.sync_copy(cy.at[b], yv)
    pltpu.sync_copy(cz.at[b], zv)
    pltpu.sync_copy(num.at[b], numv)
    pltpu.sync_copy(nbr.at[b, pl.ds(i0 * MM, PPT)], nbrv)
    pltpu.sync_copy(rcov96, rtab)
    pltpu.sync_copy(cnmax96, ctab)

    iota = lax.iota(jnp.int32, 16)
    iota64 = iota * MM

    def fill_rcovn(t, carry):
        z16 = numv[pl.ds(t * 16, 16)]
        rcovn[pl.ds(t * 16, 16)] = plsc.load_gather(rtab, [z16])
        return carry

    lax.fori_loop(0, NN // 16, fill_rcovn, 0)

    def atom_body(a, carry):
        base = a * 16
        gbase = i0 + base
        ivec = gbase + iota
        xi = xv[pl.ds(gbase, 16)]
        yi = yv[pl.ds(gbase, 16)]
        zi = zv[pl.ds(gbase, 16)]
        rci = rcovn[pl.ds(gbase, 16)]

        def m_body(m, acc):
            idx = iota64 + (base * MM + m)
            j = plsc.load_gather(nbrv, [idx])
            xj = plsc.load_gather(xv, [j])
            yj = plsc.load_gather(yv, [j])
            zj = plsc.load_gather(zv, [j])
            rcj = plsc.load_gather(rcovn, [j])
            dx = xi - xj
            dy = yi - yj
            dz = zi - zj
            d2 = jnp.maximum(dx * dx + dy * dy + dz * dz, 1e-12)
            inv_d = _rsqrt(d2)
            t = K1 * ((rci + rcj) * inv_d * (1.0 / BOHR_INV) - 1.0)
            cnm = 1.0 / (1.0 + jnp.exp(t))
            return acc + jnp.where(j != ivec, cnm, 0.0)

        acc = lax.fori_loop(0, MM, m_body, jnp.zeros((16,), jnp.float32))
        z16 = numv[pl.ds(gbase, 16)]
        cm = plsc.load_gather(ctab, [z16])
        cnout[pl.ds(base, 16)] = jnp.minimum(acc, cm)
        return carry

    lax.fori_loop(0, APT // 16, atom_body, 0)
    pltpu.sync_copy(cnout, cn_out.at[b, pl.ds(i0, APT)])


@functools.partial(
    pl.kernel,
    mesh=_mesh,
    out_type=jax.ShapeDtypeStruct((BN, 64), jnp.float32),
    compiler_params=_params,
    scratch_types=[
        pltpu.VMEM((NN,), jnp.float32),
        pltpu.VMEM((NN,), jnp.float32),
        pltpu.VMEM((NN,), jnp.float32),
        pltpu.VMEM((NN,), jnp.int32),
        pltpu.VMEM((NN,), jnp.float32),
        pltpu.VMEM((NN,), jnp.float32),
        pltpu.VMEM((PPT,), jnp.int32),
        pltpu.VMEM((96,), jnp.float32),
        pltpu.VMEM((CHUNK,), jnp.int32),
        pltpu.VMEM((CHUNK,), jnp.int32),
        pltpu.VMEM((CHUNK, ROWW), jnp.float32),
        pltpu.VMEM((CHUNK, ROWW), jnp.float32),
        pltpu.VMEM((16,), jnp.float32),
        pltpu.SemaphoreType.DMA,
        pltpu.SemaphoreType.DMA,
    ],
)
def _energy_kernel(cx, cy, cz, num, nbr, cnarr, rr96, table, out,
                   xv, yv, zv, numv, cnv, rrv, nbrv, rtab, idxb0, idxb1,
                   rows0, rows1, accv, sem0, sem1):
    c = lax.axis_index("c")
    s = lax.axis_index("s")
    wid = c * 16 + s
    b = wid // 4
    i0 = (wid % 4) * APT

    pltpu.sync_copy(cx.at[b], xv)
    pltpu.sync_copy(cy.at[b], yv)
    pltpu.sync_copy(cz.at[b], zv)
    pltpu.sync_copy(num.at[b], numv)
    pltpu.sync_copy(cnarr.at[b], cnv)
    pltpu.sync_copy(nbr.at[b, pl.ds(i0 * MM, PPT)], nbrv)
    pltpu.sync_copy(rr96, rtab)

    iota = lax.iota(jnp.int32, 16)

    def fill_rrv(t, carry):
        z16 = numv[pl.ds(t * 16, 16)]
        rrv[pl.ds(t * 16, 16)] = plsc.load_gather(rtab, [z16])
        return carry

    lax.fori_loop(0, NN // 16, fill_rrv, 0)

    def fill_idx(ch, idxb):
        p0 = ch * CHUNK
        zivec = None
        for v in range(8):
            jv = nbrv[pl.ds(p0 + 16 * v, 16)]
            zj = plsc.load_gather(numv, [jv])
            if v % 4 == 0:
                ig = i0 + ch * 2 + v // 4
                zivec = plsc.load_gather(numv, [jnp.full((16,), ig, jnp.int32)])
            idxb[pl.ds(16 * v, 16)] = zivec * NZ + zj

    def compute(ch, rows, acc):
        p0 = ch * CHUNK
        ig = None
        cni = xi = yi = zi = rri = None
        for v in range(8):
            jv = nbrv[pl.ds(p0 + 16 * v, 16)]
            if v % 4 == 0:
                ig = i0 + ch * 2 + v // 4
                igv = jnp.full((16,), ig, jnp.int32)
                cni = plsc.load_gather(cnv, [igv])
                xi = plsc.load_gather(xv, [igv])
                yi = plsc.load_gather(yv, [igv])
                zi = plsc.load_gather(zv, [igv])
                rri = plsc.load_gather(rrv, [igv])
            cnj = plsc.load_gather(cnv, [jv])
            xj = plsc.load_gather(xv, [jv])
            yj = plsc.load_gather(yv, [jv])
            zj = plsc.load_gather(zv, [jv])
            rrj = plsc.load_gather(rrv, [jv])
            rowid = (16 * v) + iota

            def ref_body(e, wz):
                w, zacc = wz
                bv = iota + e
                c0 = jnp.where(bv >= NREF, bv - NREF, bv)
                c6 = plsc.load_gather(rows, [rowid, c0])
                ca = plsc.load_gather(rows, [rowid, c0 + NREF])
                cb = plsc.load_gather(rows, [rowid, c0 + 2 * NREF])
                da = cni - ca
                db = cnj - cb
                lv = jnp.exp(K3 * (da * da + db * db))
                return (w + lv, zacc + c6 * lv)

            w, zacc = lax.fori_loop(
                0, NREF, ref_body,
                (jnp.zeros((16,), jnp.float32),
                 jnp.zeros((16,), jnp.float32)))
            mask = jnp.logical_and(jv != ig, w >= 1e-5)
            zacc = jnp.where(mask, zacc, 0.0)
            c6ij = zacc / jnp.maximum(w, 1e-5)
            dx = xi - xj
            dy = yi - yj
            dz = zi - zj
            d2 = jnp.maximum(dx * dx + dy * dy + dz * dz, 1e-12) * (BOHR_INV * BOHR_INV)
            d6 = d2 * d2 * d2
            d8 = d6 * d2
            rrij = 3.0 * rri * rrj
            r0 = A1 * (rrij * _rsqrt(rrij)) + A2
            r02 = r0 * r0
            r06 = r02 * r02 * r02
            r08 = r06 * r02
            acc = acc + c6ij * (S6 / (d6 + r06) + S8 * rrij / (d8 + r08))
        return acc

    NT = NCHUNK // 2
    fill_idx(0, idxb0)
    pltpu.async_copy(table.at[idxb0], rows0, sem0)

    def pair_body(t, acc):
        ch0 = 2 * t
        fill_idx(ch0 + 1, idxb1)
        pltpu.async_copy(table.at[idxb1], rows1, sem1)
        pltpu.make_async_copy(table.at[idxb0], rows0, sem0).wait()
        acc = compute(ch0, rows0, acc)

        @pl.when(t + 1 < NT)
        def _():
            fill_idx(ch0 + 2, idxb0)
            pltpu.async_copy(table.at[idxb0], rows0, sem0)

        pltpu.make_async_copy(table.at[idxb1], rows1, sem1).wait()
        return compute(ch0 + 1, rows1, acc)

    acc = lax.fori_loop(0, NT, pair_body, jnp.zeros((16,), jnp.float32))

    accv[...] = acc
    pltpu.sync_copy(accv, out.at[b, pl.ds((wid % 4) * 16, 16)])


def kernel(coord, c6ab, r4r2, rcov, cnmax, numbers, nbr_idx_lr):
    coord = coord.astype(jnp.float32)
    cx = coord[:, :, 0]
    cy = coord[:, :, 1]
    cz = coord[:, :, 2]
    num = numbers.astype(jnp.int32)
    nbr = nbr_idx_lr.astype(jnp.int32).reshape(BN, NN * MM)
    pad1 = jnp.zeros((1,), jnp.float32)
    rcov96 = jnp.concatenate([rcov.astype(jnp.float32), pad1])
    cnmax96 = jnp.concatenate([cnmax.astype(jnp.float32), pad1])
    rr96 = jnp.concatenate([r4r2.astype(jnp.float32), pad1])
    c6flat = c6ab.astype(jnp.float32).reshape(NZ * NZ, NREF, 3)
    table = jnp.concatenate(
        [c6flat[:, :, 0], c6flat[:, :, 1], c6flat[:, :, 2],
         jnp.zeros((NZ * NZ, ROWW - 3 * NREF), jnp.float32)], axis=1)

    cn = _cn_kernel(cx, cy, cz, num, nbr, rcov96, cnmax96)
    partials = _energy_kernel(cx, cy, cz, num, nbr, cn, rr96, table)
    return jnp.sum(partials, axis=1) * (-HALF_HARTREE)
